# Initial kernel scaffold; baseline (speedup 1.0000x reference)
import jax
import jax.numpy as jnp
from jax.experimental import pallas as pl


def kernel(x, edge_index, edge_attrs, att, istraining):
    raise NotImplementedError("write your pallas kernel here")


# speedup vs baseline: 23.5028x; 23.5028x over previous
import functools

import jax
import jax.numpy as jnp
from jax import lax
from jax.experimental import pallas as pl
from jax.experimental.pallas import tpu as pltpu
from jax.experimental.pallas import tpu_sc as plsc

N = 50000
E = 1600000
NC = 2
NS = 16
NW = NC * NS
NPAD = 50176
PER_SC_NODES = NPAD // NS
SUB = 784
C = 256
NCHUNKS = E // C
BASE_CHUNKS = NCHUNKS // NW
EXTRA = NCHUNKS - BASE_CHUNKS * NW
OUT_PER_W = NPAD // NW

_F = jnp.float32
_I = jnp.int32
_U = jnp.uint32

PI_HI = 3.140625
PI_MID = 9.67502593994140625e-4
PI_LO = 1.509957768e-7
INV_PI = 0.3183098861837907
SIN_C = (-1.0 / 6, 1.0 / 120, -1.0 / 5040, 1.0 / 362880, -1.0 / 39916800)
COS_C = (-0.5, 1.0 / 24, -1.0 / 720, 1.0 / 40320, -1.0 / 3628800,
         1.0 / 479001600)


def _cst(v, dtype=_F):
    return jnp.full((16,), v, dtype=dtype)


def _bf16_hi(v):
    u = plsc.bitcast(v, _U)
    r = u + _cst(0x7FFF, _U) + ((u >> _cst(16, _U)) & _cst(1, _U))
    return r & _cst(0xFFFF0000, _U)


def _sincos(th):
    z = th * _cst(INV_PI)
    k = (z + jnp.where(z >= 0.0, _cst(0.5), _cst(-0.5))).astype(_I)
    kf = k.astype(_F)
    r = th - kf * _cst(PI_HI)
    r = r - kf * _cst(PI_MID)
    r = r - kf * _cst(PI_LO)
    r2 = r * r
    sp = _cst(SIN_C[4])
    for c in SIN_C[3::-1]:
        sp = sp * r2 + _cst(c)
    sinr = r + (r * r2) * sp
    cp = _cst(COS_C[5])
    for c in COS_C[4::-1]:
        cp = cp * r2 + _cst(c)
    cosr = _cst(1.0) + r2 * cp
    sgn = jnp.where((k & _cst(1, _I)) == 0, _cst(1.0), _cst(-1.0))
    return sgn * sinr, sgn * cosr


def _unpack_hi(w):
    return plsc.bitcast(plsc.bitcast(w, _U) & _cst(0xFFFF0000, _U), _F)


def _unpack_lo(w):
    return plsc.bitcast(plsc.bitcast(w, _U) << _cst(16, _U), _F)


_mesh = plsc.VectorSubcoreMesh(core_axis_name="c", subcore_axis_name="s")


@functools.partial(
    pl.kernel,
    out_type=jax.ShapeDtypeStruct((NC * 4 * NPAD,), _F),
    mesh=_mesh,
    compiler_params=pltpu.CompilerParams(needs_layout_passes=False),
    scratch_types=[
        pltpu.VMEM((NPAD,), _I),
        pltpu.VMEM((NPAD,), _I),
        pltpu.VMEM((C * 20,), _F),
        pltpu.VMEM((C,), _I),
        pltpu.VMEM((2, 128), _I),
        pltpu.VMEM((C,), _F),
        pltpu.VMEM((C,), _F),
        pltpu.VMEM((C,), _F),
        pltpu.VMEM((C,), _F),
        pltpu.VMEM((6, 16), _F),
        pltpu.VMEM((SUB,), _F),
        pltpu.VMEM_SHARED((NPAD,), _I),
        pltpu.VMEM_SHARED((NPAD,), _I),
        pltpu.VMEM_SHARED((NPAD,), _F),
        pltpu.VMEM_SHARED((NPAD,), _F),
        pltpu.VMEM_SHARED((NPAD,), _F),
        pltpu.VMEM_SHARED((NPAD,), _F),
        pltpu.SemaphoreType.DMA,
    ],
)
def _edge_kernel(x_hbm, src_hbm, dst_hbm, ea_hbm, attb_hbm, part_hbm,
                 xp01_v, xp2s_v, eav, srcv, dstv, od, o0, o1, o2, attv,
                 zbuf,
                 xp01_sh, xp2s_sh, acc_d, acc_0, acc_1, acc_2, sem):
    cid = lax.axis_index("c")
    sid = lax.axis_index("s")
    wid = cid * NS + sid
    iota = lax.iota(_I, 16)

    pltpu.sync_copy(attb_hbm, attv)
    a0, a1, a2 = attv[0], attv[1], attv[2]
    t0, t1, t2 = attv[3], attv[4], attv[5]

    def _zb(i, _):
        zbuf[pl.ds(i * 16, 16)] = _cst(0.0)
        return _
    lax.fori_loop(0, SUB // 16, _zb, None)
    nb0 = sid * PER_SC_NODES
    for q in range(PER_SC_NODES // SUB):
        for acc in (acc_d, acc_0, acc_1, acc_2):
            pltpu.sync_copy(zbuf, acc.at[pl.ds(nb0 + q * SUB, SUB)])

    iota3 = iota * 3
    for q in range(PER_SC_NODES // SUB):
        nb = nb0 + q * SUB
        pltpu.sync_copy(x_hbm.at[pl.ds(nb * 3, SUB * 3)],
                        eav.at[pl.ds(0, SUB * 3)])

        def _pk(i, _):
            ridx = iota3 + i * 48
            x0 = plsc.load_gather(eav, [ridx])
            x1 = plsc.load_gather(eav, [ridx + _cst(1, _I)])
            x2 = plsc.load_gather(eav, [ridx + _cst(2, _I)])
            s = x0 * a0 + x1 * a1 + x2 * a2
            w01 = _bf16_hi(x0) | (_bf16_hi(x1) >> _cst(16, _U))
            w2s = _bf16_hi(x2) | (_bf16_hi(s) >> _cst(16, _U))
            xp01_v[pl.ds(nb + i * 16, 16)] = plsc.bitcast(w01, _I)
            xp2s_v[pl.ds(nb + i * 16, 16)] = plsc.bitcast(w2s, _I)
            return _
        lax.fori_loop(0, SUB // 16, _pk, None)
        pltpu.sync_copy(xp01_v.at[pl.ds(nb, SUB)], xp01_sh.at[pl.ds(nb, SUB)])
        pltpu.sync_copy(xp2s_v.at[pl.ds(nb, SUB)], xp2s_sh.at[pl.ds(nb, SUB)])

    plsc.subcore_barrier()
    pltpu.sync_copy(xp01_sh, xp01_v)
    pltpu.sync_copy(xp2s_sh, xp2s_v)

    n_chunks = BASE_CHUNKS + jnp.where(wid < EXTRA, 1, 0).astype(_I)

    iota20 = iota * 20

    def _chunk(i, _):
        ebase = (i * NW + wid) * C
        pltpu.sync_copy(src_hbm.at[pl.ds(ebase, C)], srcv)
        pltpu.sync_copy(dst_hbm.at[pl.ds(ebase, 128)], dstv.at[0])
        pltpu.sync_copy(dst_hbm.at[pl.ds(ebase + 128, 128)], dstv.at[1])
        pltpu.sync_copy(ea_hbm.at[pl.ds(ebase * 20, C * 20)], eav)

        for j in range(C // 16):
            sb = j * 16
            sidx = srcv[pl.ds(sb, 16)]
            didx = dstv[j // 8, pl.ds((j % 8) * 16, 16)]
            w01 = plsc.load_gather(xp01_v, [sidx])
            w2s = plsc.load_gather(xp2s_v, [sidx])
            wds = plsc.load_gather(xp2s_v, [didx])
            x0 = _unpack_hi(w01)
            x1 = _unpack_lo(w01)
            x2 = _unpack_hi(w2s)
            s = _unpack_lo(wds)
            ridx = iota20 + sb * 20
            th = plsc.load_gather(eav, [ridx + _cst(9, _I)])
            e10 = plsc.load_gather(eav, [ridx + _cst(11, _I)])
            e11 = plsc.load_gather(eav, [ridx + _cst(12, _I)])
            e12 = plsc.load_gather(eav, [ridx + _cst(13, _I)])
            e20 = plsc.load_gather(eav, [ridx + _cst(14, _I)])
            e21 = plsc.load_gather(eav, [ridx + _cst(15, _I)])
            e22 = plsc.load_gather(eav, [ridx + _cst(16, _I)])
            e30 = plsc.load_gather(eav, [ridx + _cst(17, _I)])
            e31 = plsc.load_gather(eav, [ridx + _cst(18, _I)])
            e32 = plsc.load_gather(eav, [ridx + _cst(19, _I)])
            a = x0 * e10 + x1 * e11 + x2 * e12
            b = x0 * e20 + x1 * e21 + x2 * e22
            sn, cs = _sincos(th)
            ac = a * cs
            asn = a * sn
            pt0 = ac * e10 + asn * e30 + b * e20
            pt1 = ac * e11 + asn * e31 + b * e21
            pt2 = ac * e12 + asn * e32 + b * e22
            lg = s + t0 * pt0 + t1 * pt1 + t2 * pt2
            lg = jnp.where(lg > 0.0, lg, lg * 0.2)
            p = jnp.exp(lg)
            od[pl.ds(sb, 16)] = p
            o0[pl.ds(sb, 16)] = p * pt0
            o1[pl.ds(sb, 16)] = p * pt1
            o2[pl.ds(sb, 16)] = p * pt2

        for r in range(2):
            idx = dstv.at[r]
            sl = pl.ds(r * 128, 128)
            pltpu.sync_copy(od.at[sl], acc_d.at[idx], add=True)
            pltpu.sync_copy(o0.at[sl], acc_0.at[idx], add=True)
            pltpu.sync_copy(o1.at[sl], acc_1.at[idx], add=True)
            pltpu.sync_copy(o2.at[sl], acc_2.at[idx], add=True)
        return _

    lax.fori_loop(0, n_chunks, _chunk, None)

    plsc.subcore_barrier()
    pbase = cid * 4 * NPAD + nb0
    for k, acc in enumerate((acc_d, acc_0, acc_1, acc_2)):
        for q in range(PER_SC_NODES // SUB):
            pltpu.sync_copy(acc.at[pl.ds(nb0 + q * SUB, SUB)], zbuf)
            pltpu.sync_copy(zbuf, part_hbm.at[pl.ds(pbase + k * NPAD
                                                    + q * SUB, SUB)])


@functools.partial(
    pl.kernel,
    out_type=jax.ShapeDtypeStruct((NPAD * 3,), _F),
    mesh=_mesh,
    compiler_params=pltpu.CompilerParams(needs_layout_passes=False),
    scratch_types=[
        pltpu.VMEM((4 * OUT_PER_W,), _F),
        pltpu.VMEM((4 * OUT_PER_W,), _F),
        pltpu.VMEM((OUT_PER_W * 3,), _F),
    ],
)
def _finalize_kernel(part_hbm, out_hbm, pa, pb, stage):
    cid = lax.axis_index("c")
    sid = lax.axis_index("s")
    wid = cid * NS + sid
    nb = wid * OUT_PER_W
    iota = lax.iota(_I, 16)
    for k in range(4):
        pltpu.sync_copy(part_hbm.at[pl.ds(k * NPAD + nb, OUT_PER_W)],
                        pa.at[pl.ds(k * OUT_PER_W, OUT_PER_W)])
        pltpu.sync_copy(part_hbm.at[pl.ds(4 * NPAD + k * NPAD + nb,
                                          OUT_PER_W)],
                        pb.at[pl.ds(k * OUT_PER_W, OUT_PER_W)])

    iota3 = iota * 3

    def _fin(i, _):
        d = pa[pl.ds(i * 16, 16)] + pb[pl.ds(i * 16, 16)]
        inv = _cst(1.0) / (d + _cst(1e-16))
        lidx = iota3 + i * 48
        for k in range(3):
            o = (k + 1) * OUT_PER_W
            v = (pa[pl.ds(o + i * 16, 16)] + pb[pl.ds(o + i * 16, 16)]) * inv
            plsc.store_scatter(stage, [lidx + _cst(k, _I)], v)
        return _
    lax.fori_loop(0, OUT_PER_W // 16, _fin, None)
    pltpu.sync_copy(stage, out_hbm.at[pl.ds(nb * 3, OUT_PER_W * 3)])


def kernel(x, edge_index, edge_attrs, att, istraining):
    xpad = jnp.concatenate(
        [x.astype(_F), jnp.zeros((NPAD - N, 3), _F)], axis=0).reshape(-1)
    attb = jnp.broadcast_to(
        att.astype(_F)[0][:, None], (6, 16))
    ei = edge_index.astype(_I)
    part = _edge_kernel(xpad, ei[0], ei[1],
                        edge_attrs.astype(_F).reshape(-1), attb)
    out = _finalize_kernel(part)
    return out.reshape(NPAD, 3)[:N]
